# baseline (speedup 1.0000x reference)
import jax
import jax.numpy as jnp
from jax.experimental import pallas as pl
from jax.experimental.pallas import tpu as pltpu

_BM = 256
_K = 8192
_ED = 32

_DEF = jax.lax.Precision.DEFAULT
_HI = jax.lax.Precision.HIGHEST


def _dotT(a, b, prec):
    return jax.lax.dot_general(a, b, (((1,), (1,)), ((), ())),
                               precision=prec,
                               preferred_element_type=jnp.float32)


def _dot(a, b, prec):
    return jax.lax.dot_general(a, b, (((1,), (0,)), ((), ())),
                               precision=prec,
                               preferred_element_type=jnp.float32)


def _tok_kernel(x_ref, W1_ref, b1_ref, W2_ref, b2_ref, W3_ref, b3_ref,
                cb0_ref, cb1_ref, cb2_ref, sem_ref, norms_ref):
    cbs = (cb0_ref, cb1_ref, cb2_ref)

    @pl.when(pl.program_id(0) == 0)
    def _():
        ones = jnp.ones((1, _ED), jnp.float32)
        for L, cb in enumerate(cbs):
            c = cb[...]
            norms_ref[L:L + 1, :] = _dotT(ones, c * c, _HI)

    h = _dot(x_ref[...], W1_ref[...], _DEF) + b1_ref[...]
    h = h * jax.nn.sigmoid(h)
    h = _dot(h, W2_ref[...], _DEF) + b2_ref[...]
    h = h * jax.nn.sigmoid(h)
    res = _dot(h, W3_ref[...], _DEF) + b3_ref[...]

    iota = jax.lax.broadcasted_iota(jnp.int32, (_BM, _K), 1)
    cols = []
    for L, cb in enumerate(cbs):
        C = cb[...]
        ssq = jnp.sum(res * res, axis=1, keepdims=True)
        scores = ssq - 2.0 * _dotT(res, C, _DEF) + norms_ref[L:L + 1, :]
        mins = jnp.min(scores, axis=1, keepdims=True)
        idx = jnp.min(jnp.where(scores == mins, iota, _K), axis=1,
                      keepdims=True)
        cols.append(idx)
        if L < 2:
            onehot = (iota == idx).astype(jnp.float32)
            res = res - _dot(onehot, C, _HI)
    sem_ref[...] = jnp.concatenate(cols, axis=1)


def kernel(x, W1, b1, W2, b2, W3, b3, cb0, cb1, cb2, ids, ids_fut,
           seq_mask, user_ids):
    M = x.shape[0]
    D = 3
    B, N = ids.shape
    grid = (M // _BM,)

    fixed = lambda i: (0, 0)
    sem_ids = pl.pallas_call(
        _tok_kernel,
        grid=grid,
        in_specs=[
            pl.BlockSpec((_BM, x.shape[1]), lambda i: (i, 0)),
            pl.BlockSpec(W1.shape, fixed),
            pl.BlockSpec((1, b1.shape[0]), fixed),
            pl.BlockSpec(W2.shape, fixed),
            pl.BlockSpec((1, b2.shape[0]), fixed),
            pl.BlockSpec(W3.shape, fixed),
            pl.BlockSpec((1, b3.shape[0]), fixed),
            pl.BlockSpec(cb0.shape, fixed),
            pl.BlockSpec(cb1.shape, fixed),
            pl.BlockSpec(cb2.shape, fixed),
        ],
        out_specs=pl.BlockSpec((_BM, D), lambda i: (i, 0)),
        out_shape=jax.ShapeDtypeStruct((M, D), jnp.int32),
        scratch_shapes=[pltpu.VMEM((D, _K), jnp.float32)],
        compiler_params=pltpu.CompilerParams(
            dimension_semantics=("arbitrary",)),
    )(x, W1, b1.reshape(1, -1), W2, b2.reshape(1, -1), W3,
      b3.reshape(1, -1), cb0, cb1, cb2)

    tt = jnp.arange(D, dtype=jnp.int32)
    token_type_ids = jnp.tile(tt, (B, N))
    token_type_ids_fut = jnp.tile(tt, (B, 1))
    return (user_ids, sem_ids, token_type_ids, token_type_ids_fut)


# speedup vs baseline: 3.2455x; 3.2455x over previous
import jax
import jax.numpy as jnp
from jax.experimental import pallas as pl
from jax.experimental.pallas import tpu as pltpu

_BM = 256
_K = 8192
_ED = 32

_DEF = jax.lax.Precision.DEFAULT
_HI = jax.lax.Precision.HIGHEST


def _dotT(a, b, prec):
    return jax.lax.dot_general(a, b, (((1,), (1,)), ((), ())),
                               precision=prec,
                               preferred_element_type=jnp.float32)


def _dot(a, b, prec):
    return jax.lax.dot_general(a, b, (((1,), (0,)), ((), ())),
                               precision=prec,
                               preferred_element_type=jnp.float32)


_NHI = _K // _ED


def _gather_rows(idx, cbr):
    hi = idx // _ED
    lo = idx - hi * _ED
    iota_hi = jax.lax.broadcasted_iota(jnp.int32, (_BM, _NHI), 1)
    oh = (iota_hi == hi).astype(jnp.float32)
    g1 = _dot(oh, cbr, _HI)
    nb = (_K // _NHI) * _ED
    iota_b = jax.lax.broadcasted_iota(jnp.int32, (_BM, nb), 1)
    mask = ((iota_b // _ED) == lo).astype(jnp.float32)
    fold = (jax.lax.broadcasted_iota(jnp.int32, (nb, _ED), 0) % _ED ==
            jax.lax.broadcasted_iota(jnp.int32, (nb, _ED), 1)).astype(jnp.float32)
    return _dot(mask * g1, fold, _HI)


def _tok_kernel(x_ref, W1_ref, b1_ref, W2_ref, b2_ref, W3_ref, b3_ref,
                cb0_ref, cb1_ref, cb2_ref, cb0r_ref, cb1r_ref,
                sem_ref, norms_ref):
    cbs = (cb0_ref, cb1_ref, cb2_ref)
    cbrs = (cb0r_ref, cb1r_ref)

    @pl.when(pl.program_id(0) == 0)
    def _():
        ones = jnp.ones((1, _ED), jnp.float32)
        for L, cb in enumerate(cbs):
            c = cb[...]
            norms_ref[L:L + 1, :] = _dotT(ones, c * c, _HI)

    h = _dot(x_ref[...], W1_ref[...], _DEF) + b1_ref[...]
    h = h * jax.nn.sigmoid(h)
    h = _dot(h, W2_ref[...], _DEF) + b2_ref[...]
    h = h * jax.nn.sigmoid(h)
    res = _dot(h, W3_ref[...], _DEF) + b3_ref[...]

    iota = jax.lax.broadcasted_iota(jnp.int32, (_BM, _K), 1)
    cols = []
    for L, cb in enumerate(cbs):
        C = cb[...]
        ssq = jnp.sum(res * res, axis=1, keepdims=True)
        scores = ssq - 2.0 * _dotT(res, C, _DEF) + norms_ref[L:L + 1, :]
        mins = jnp.min(scores, axis=1, keepdims=True)
        idx = jnp.min(jnp.where(scores == mins, iota, _K), axis=1,
                      keepdims=True)
        cols.append(idx)
        if L < 2:
            res = res - _gather_rows(idx, cbrs[L][...])
    sem_ref[...] = jnp.concatenate(cols, axis=1)


def kernel(x, W1, b1, W2, b2, W3, b3, cb0, cb1, cb2, ids, ids_fut,
           seq_mask, user_ids):
    M = x.shape[0]
    D = 3
    B, N = ids.shape
    grid = (M // _BM,)

    fixed = lambda i: (0, 0)
    sem_ids = pl.pallas_call(
        _tok_kernel,
        grid=grid,
        in_specs=[
            pl.BlockSpec((_BM, x.shape[1]), lambda i: (i, 0)),
            pl.BlockSpec(W1.shape, fixed),
            pl.BlockSpec((1, b1.shape[0]), fixed),
            pl.BlockSpec(W2.shape, fixed),
            pl.BlockSpec((1, b2.shape[0]), fixed),
            pl.BlockSpec(W3.shape, fixed),
            pl.BlockSpec((1, b3.shape[0]), fixed),
            pl.BlockSpec(cb0.shape, fixed),
            pl.BlockSpec(cb1.shape, fixed),
            pl.BlockSpec(cb2.shape, fixed),
            pl.BlockSpec((_NHI, _K // _NHI * _ED), fixed),
            pl.BlockSpec((_NHI, _K // _NHI * _ED), fixed),
        ],
        out_specs=pl.BlockSpec((_BM, D), lambda i: (i, 0)),
        out_shape=jax.ShapeDtypeStruct((M, D), jnp.int32),
        scratch_shapes=[pltpu.VMEM((D, _K), jnp.float32)],
        compiler_params=pltpu.CompilerParams(
            dimension_semantics=("arbitrary",)),
    )(x, W1, b1.reshape(1, -1), W2, b2.reshape(1, -1), W3,
      b3.reshape(1, -1), cb0, cb1, cb2,
      cb0.reshape(_NHI, -1), cb1.reshape(_NHI, -1))

    tt = jnp.arange(D, dtype=jnp.int32)
    token_type_ids = jnp.tile(tt, (B, N))
    token_type_ids_fut = jnp.tile(tt, (B, 1))
    return (user_ids, sem_ids, token_type_ids, token_type_ids_fut)
